# baseline (speedup 1.0000x reference)
import functools

import jax
import jax.numpy as jnp
from jax import lax
from jax.experimental import pallas as pl
from jax.experimental.pallas import tpu as pltpu
from jax.experimental.pallas import tpu_sc as plsc

F32 = jnp.float32

N0 = 2048
D = 128
KK1, P1 = 1843, 1920
KK2, P2 = 1290, 1536


def _gcn(A, h_in, Wt, b, *, transpose_lhs=False, deg=None, scale=None,
         resid=None, resid2=None, pool=None, score_kk=0, mask_rows=0):
    rows = A.shape[1] if transpose_lhs else A.shape[0]
    has_deg = deg is not None
    has_scale = scale is not None
    has_resid = resid is not None
    has_resid2 = resid2 is not None
    has_pool = pool is not None

    def body(*refs):
        it = iter(refs)
        a_ref = next(it)
        h_ref = next(it)
        wt_ref = next(it)
        b_ref = next(it)
        deg_ref = next(it) if has_deg else None
        sc_ref = next(it) if has_scale else None
        r_ref = next(it) if has_resid else None
        r2_ref = next(it) if has_resid2 else None
        p_ref = next(it) if has_pool else None
        pb_ref = next(it) if has_pool else None
        o_ref = next(it)
        o2_ref = next(it) if has_resid2 else None
        s_ref = next(it) if has_pool else None

        hv = h_ref[...]
        if has_scale:
            hv = hv * sc_ref[...]
        if transpose_lhs:
            t = lax.dot_general(a_ref[...], hv, (((0,), (0,)), ((), ())),
                                preferred_element_type=F32)
        else:
            t = jnp.dot(a_ref[...], hv, preferred_element_type=F32)
        if has_deg:
            t = t / deg_ref[...]
        out = jnp.maximum(jnp.dot(t, wt_ref[...], preferred_element_type=F32)
                          + b_ref[...], 0.0)
        if has_resid:
            out = out + r_ref[...]
        if mask_rows:
            ri = lax.broadcasted_iota(jnp.int32, out.shape, 0)
            out = jnp.where(ri < mask_rows, out, 0.0)
        o_ref[...] = out
        if has_resid2:
            o2_ref[...] = out + r2_ref[...]
        if has_pool:
            s = jnp.sum(out * p_ref[...], axis=1, keepdims=True) + pb_ref[...]
            s = jax.nn.sigmoid(s)
            if score_kk:
                ri1 = lax.broadcasted_iota(jnp.int32, s.shape, 0)
                s = jnp.where(ri1 < score_kk, s, -1.0)
            s_ref[...] = s

    inputs = [A, h_in, Wt, b]
    if has_deg:
        inputs.append(deg)
    if has_scale:
        inputs.append(scale)
    if has_resid:
        inputs.append(resid)
    if has_resid2:
        inputs.append(resid2)
    if has_pool:
        inputs.extend(pool)
    out_shape = [jax.ShapeDtypeStruct((rows, D), F32)]
    if has_resid2:
        out_shape.append(jax.ShapeDtypeStruct((rows, D), F32))
    if has_pool:
        out_shape.append(jax.ShapeDtypeStruct((rows, 1), F32))
    res = pl.pallas_call(body, out_shape=out_shape)(*inputs)
    return res[0] if len(res) == 1 else tuple(res)


def _adj_pool(Crows, Drows, kk):
    P, M = Crows.shape
    BLK = 128

    def body(c_ref, d_ref, b_ref, deg_ref):
        i = pl.program_id(0)
        cb = (c_ref[...] != 0).astype(jnp.bfloat16)
        db = (d_ref[...] != 0).astype(jnp.bfloat16)
        cnt = lax.dot_general(cb, db, (((1,), (1,)), ((), ())),
                              preferred_element_type=F32)
        ri = i * BLK + lax.broadcasted_iota(jnp.int32, (BLK, P), 0)
        ci = lax.broadcasted_iota(jnp.int32, (BLK, P), 1)
        Bv = jnp.where((cnt != 0) & (ri < kk) & (ci < kk), 1.0, 0.0)
        b_ref[...] = Bv
        rs = jnp.sum(Bv, axis=1, keepdims=True)
        ri1 = i * BLK + lax.broadcasted_iota(jnp.int32, (BLK, 1), 0)
        deg_ref[...] = jnp.where(ri1 < kk, rs, 1.0)

    return pl.pallas_call(
        body,
        grid=(P // BLK,),
        in_specs=[pl.BlockSpec((BLK, M), lambda i: (i, 0)),
                  pl.BlockSpec((P, M), lambda i: (0, 0))],
        out_specs=[pl.BlockSpec((BLK, P), lambda i: (i, 0)),
                   pl.BlockSpec((BLK, 1), lambda i: (i, 0))],
        out_shape=[jax.ShapeDtypeStruct((P, P), F32),
                   jax.ShapeDtypeStruct((P, 1), F32)],
    )(Crows, Drows)


def _rank(s_col, s_row):
    P = s_col.shape[0]
    BLK = 128

    def body(sc_ref, sr_ref, o_ref):
        i = pl.program_id(0)
        sc = sc_ref[...]
        sr = sr_ref[...]
        ri = i * BLK + lax.broadcasted_iota(jnp.int32, (BLK, P), 0)
        ci = lax.broadcasted_iota(jnp.int32, (BLK, P), 1)
        before = (sr > sc) | ((sr == sc) & (ci < ri))
        o_ref[...] = jnp.sum(before.astype(F32), axis=1, keepdims=True)

    return pl.pallas_call(
        body,
        grid=(P // BLK,),
        in_specs=[pl.BlockSpec((BLK, 1), lambda i: (i, 0)),
                  pl.BlockSpec((1, P), lambda i: (0, 0))],
        out_specs=pl.BlockSpec((BLK, 1), lambda i: (i, 0)),
        out_shape=jax.ShapeDtypeStruct((P, 1), F32),
    )(s_col, s_row)


def _select(rank_row, s_row):
    P = rank_row.shape[1]
    BLK = 128

    def body(r_ref, s_ref, i_ref, v_ref):
        i = pl.program_id(0)
        rr = r_ref[...]
        sr = s_ref[...]
        rg = (i * BLK + lax.broadcasted_iota(jnp.int32, (BLK, P), 0)).astype(F32)
        match = (rr == rg).astype(F32)
        ci = lax.broadcasted_iota(jnp.int32, (BLK, P), 1).astype(F32)
        i_ref[...] = jnp.sum(match * ci, axis=1, keepdims=True).astype(jnp.int32)
        v_ref[...] = jnp.sum(match * sr, axis=1, keepdims=True)

    return pl.pallas_call(
        body,
        grid=(P // BLK,),
        in_specs=[pl.BlockSpec((1, P), lambda i: (0, 0)),
                  pl.BlockSpec((1, P), lambda i: (0, 0))],
        out_specs=[pl.BlockSpec((BLK, 1), lambda i: (i, 0)),
                   pl.BlockSpec((BLK, 1), lambda i: (i, 0))],
        out_shape=[jax.ShapeDtypeStruct((P, 1), jnp.int32),
                   jax.ShapeDtypeStruct((P, 1), F32)],
    )(rank_row, s_row)


def _transpose(x):
    P = x.shape[0]
    BLK = 128

    def body(x_ref, o_ref):
        o_ref[...] = x_ref[...].T

    return pl.pallas_call(
        body,
        grid=(P // BLK, P // BLK),
        in_specs=[pl.BlockSpec((BLK, BLK), lambda i, j: (i, j))],
        out_specs=pl.BlockSpec((BLK, BLK), lambda i, j: (j, i)),
        out_shape=jax.ShapeDtypeStruct((P, P), x.dtype),
    )(x)


def _gather_rows(table, idx, B, CH):
    _, Dw = table.shape
    info = plsc.get_sparse_core_info()
    NW = info.num_cores * info.num_subcores
    b_per_w = B // NW
    mesh = plsc.VectorSubcoreMesh(core_axis_name="c", subcore_axis_name="s")

    @functools.partial(
        pl.kernel, mesh=mesh,
        out_type=jax.ShapeDtypeStruct((B, Dw), table.dtype),
        scratch_types=[pltpu.VMEM((CH,), jnp.int32),
                       pltpu.VMEM((CH, Dw), table.dtype),
                       pltpu.SemaphoreType.DMA],
    )
    def k(table_hbm, idx_hbm, out_hbm, idx_v, rows_v, sem):
        wid = lax.axis_index("s") * info.num_cores + lax.axis_index("c")
        base = wid * b_per_w
        for c in range(b_per_w // CH):
            off = base + c * CH
            pltpu.sync_copy(idx_hbm.at[pl.ds(off, CH)], idx_v)
            pltpu.async_copy(table_hbm.at[idx_v], rows_v, sem).wait()
            pltpu.sync_copy(rows_v, out_hbm.at[pl.ds(off, CH)])

    return k(table, idx)


def kernel(g, h, params):
    p = params
    W0t = p["down_W"][0].T
    W1t = p["down_W"][1].T
    Wbt = p["bottom_W"].T
    Wu0t = p["up_W"][0].T
    Wu1t = p["up_W"][1].T
    b0 = p["down_b"][0].reshape(1, D)
    b1 = p["down_b"][1].reshape(1, D)
    bb = p["bottom_b"].reshape(1, D)
    bu0 = p["up_b"][0].reshape(1, D)
    bu1 = p["up_b"][1].reshape(1, D)
    p0 = p["pool_W"][0].reshape(1, D)
    p1 = p["pool_W"][1].reshape(1, D)
    pb0 = p["pool_b"][0].reshape(1, 1)
    pb1 = p["pool_b"][1].reshape(1, 1)

    h1, s1 = _gcn(g, h, W0t, b0, pool=(p0, pb0))
    rank1 = _rank(s1, s1.reshape(1, N0))
    idxs1, vals1 = _select(rank1.reshape(1, N0), s1.reshape(1, N0))
    idx1f = idxs1.reshape(N0)

    gT = _transpose(g)
    G_r = _gather_rows(g, idx1f, N0, 16)
    GT_r = _gather_rows(gT, idx1f, N0, 16)
    h1_r = _gather_rows(h1, idx1f, N0, 64)

    B1, deg1 = _adj_pool(G_r[:P1], GT_r[:P1], KK1)

    h2, s2 = _gcn(B1, h1_r[:P1], W1t, b1, deg=deg1, scale=vals1[:P1],
                  pool=(p1, pb1), score_kk=KK1)
    rank2 = _rank(s2, s2.reshape(1, P1))
    idxs2, vals2 = _select(rank2.reshape(1, P1), s2.reshape(1, P1))
    idx2f = idxs2.reshape(P1)

    B1T = _transpose(B1)
    B1_r = _gather_rows(B1, idx2f, P2, 16)
    B1T_r = _gather_rows(B1T, idx2f, P2, 16)
    h2_r = _gather_rows(h2, idx2f, P2, 48)

    B2, deg2 = _adj_pool(B1_r, B1T_r, KK2)

    hb = _gcn(B2, h2_r, Wbt, bb, deg=deg2, scale=vals2[:P2], mask_rows=KK2)

    h_u0 = _gcn(B1T_r, hb, Wu0t, bu0, transpose_lhs=True, deg=deg1,
                resid=h2, mask_rows=KK1)

    h_u1, h_fin = _gcn(GT_r[:P1], h_u0, Wu1t, bu1, transpose_lhs=True,
                       resid=h1, resid2=h)

    return (h_u0[:KK1], h_u1, h_fin)
